# baseline (speedup 1.0000x reference)
import functools

import jax
import jax.numpy as jnp
from jax import lax
from jax.experimental import pallas as pl
from jax.experimental.pallas import tpu as pltpu
from jax.experimental.pallas import tpu_sc as plsc

N, D, K, OUT = 160000, 256, 128, 64
NC, NS = 2, 16
NW = NC * NS
SPW = K // NW
CH = 128
NV = D // 16
DEC = 8
ND = N // DEC

_mesh = plsc.VectorSubcoreMesh(core_axis_name="c", subcore_axis_name="s",
                               num_cores=NC, num_subcores=NS)


@functools.partial(
    pl.kernel,
    out_type=jax.ShapeDtypeStruct((NW, SPW, D), jnp.float32),
    mesh=_mesh,
    compiler_params=pltpu.CompilerParams(needs_layout_passes=False),
    scratch_types=[
        pltpu.VMEM((CH, D), jnp.float32),
        pltpu.VMEM((CH, D), jnp.float32),
        pltpu.VMEM((ND + 16,), jnp.int32),
        pltpu.VMEM((SPW + 1, 16), jnp.int32),
        pltpu.VMEM((SPW, D), jnp.float32),
        pltpu.SemaphoreType.DMA,
        pltpu.SemaphoreType.DMA,
    ],
)
def _sc_segment_sum(feat_hbm, ids_hbm, idsd_hbm, fsum_hbm,
                    fbuf0, fbuf1, idsv, wbuf, ostage, sem0, sem1):
  c = lax.axis_index("c")
  sub = lax.axis_index("s")
  wid = c * NS + sub
  fbufs = (fbuf0, fbuf1)
  sems = (sem0, sem1)

  pltpu.sync_copy(idsd_hbm, idsv.at[pl.ds(0, ND)])
  woffs = []
  for s in range(SPW + 1):
    k = SPW * wid + s

    def bs_body(_, lohi, k=k):
      lo, hi = lohi
      mid = (lo + hi) >> 1
      v = idsv[pl.ds(mid, 16)][0]
      return jnp.where(v < k, mid + 1, lo), jnp.where(v < k, hi, mid)

    jl, _ = lax.fori_loop(0, 15, bs_body, (jnp.int32(0), jnp.int32(ND)))
    woff = pl.multiple_of(jnp.clip(DEC * jl - DEC, 0, N - 16), 8)
    woffs.append(woff)
    pltpu.async_copy(ids_hbm.at[pl.ds(woff, 16)], wbuf.at[s], sem0)
  for s in range(SPW + 1):
    pltpu.make_async_copy(ids_hbm.at[pl.ds(woffs[s], 16)],
                          wbuf.at[s], sem0).wait()
  bnd = []
  for s in range(SPW + 1):
    k = SPW * wid + s
    cnt = plsc.all_reduce_population_count(wbuf[s] < k)[0]
    bnd.append(woffs[s] + cnt)

  for s in range(SPW):
    for t in range(NV):
      ostage[s, pl.ds(16 * t, 16)] = jnp.zeros((16,), jnp.float32)

  a8 = (bnd[0] >> 3) << 3
  nch = lax.div(bnd[SPW] - a8 + (CH - 1), CH)

  def chunk_start(g, slot):
    @pl.when(g < nch)
    def _():
      cs = pl.multiple_of(jnp.minimum(a8 + g * CH, N - CH), 8)
      pltpu.async_copy(feat_hbm.at[pl.ds(cs, CH)], fbufs[slot], sems[slot])

  def chunk_wait(g, slot):
    @pl.when(g < nch)
    def _():
      cs = pl.multiple_of(jnp.minimum(a8 + g * CH, N - CH), 8)
      pltpu.make_async_copy(feat_hbm.at[pl.ds(cs, CH)],
                            fbufs[slot], sems[slot]).wait()

  def chunk_rows(g, slot):
    cs0 = a8 + g * CH
    cs = jnp.minimum(cs0, N - CH)
    fb = fbufs[slot]
    for s in range(SPW):
      lo = jnp.maximum(bnd[s], cs0) - cs
      hi = jnp.minimum(bnd[s + 1], cs0 + CH) - cs

      @pl.when(lo < hi)
      def _(s=s, lo=lo, hi=hi):
        n4 = lo + (((hi - lo) >> 2) << 2)

        def quad_body(r0, accs):
          r = lo + 4 * r0
          return tuple(
              ((accs[t] + fb[r, pl.ds(16 * t, 16)]
                + fb[r + 1, pl.ds(16 * t, 16)])
               + (fb[r + 2, pl.ds(16 * t, 16)]
                  + fb[r + 3, pl.ds(16 * t, 16)]))
              for t in range(NV))

        def row_body(r, accs):
          return tuple(accs[t] + fb[r, pl.ds(16 * t, 16)] for t in range(NV))

        carry = lax.fori_loop(
            0, (hi - lo) >> 2, quad_body,
            tuple(jnp.zeros((16,), jnp.float32) for _ in range(NV)))
        carry = lax.fori_loop(n4, hi, row_body, carry)
        for t in range(NV):
          ostage[s, pl.ds(16 * t, 16)] += carry[t]

  def pair_body(j, _):
    g0 = 2 * j
    chunk_start(g0 + 1, 1)
    chunk_wait(g0, 0)
    chunk_rows(g0, 0)
    chunk_start(g0 + 2, 0)
    chunk_wait(g0 + 1, 1)
    chunk_rows(g0 + 1, 1)
    return 0

  chunk_start(0, 0)
  lax.fori_loop(0, lax.div(nch + 1, 2), pair_body, 0)

  pltpu.sync_copy(ostage, fsum_hbm.at[wid])


BN2 = 4000
NB2 = N // BN2


def _tc_aux_body(ids_ref, c3_ref, aux_ref):
  i = pl.program_id(0)

  @pl.when(i == 0)
  def _():
    aux_ref[...] = jnp.zeros_like(aux_ref)

  ids = ids_ref[0, 0]
  oh = (lax.broadcasted_iota(jnp.int32, (K, BN2), 0)
        == ids[None, :]).astype(jnp.float32)
  csum = lax.dot_general(oh, c3_ref[...], (((1,), (0,)), ((), ())),
                         preferred_element_type=jnp.float32)
  cnt = jnp.sum(oh, axis=1, keepdims=True)
  aux_ref[...] += jnp.concatenate([csum, cnt], axis=1)


def _tc_head_body(aux_ref, fs_ref, w1_ref, w2_ref, w3_ref, b3_ref,
                  emb_ref, cent_ref, out_ref):
  aux4 = aux_ref[...]
  inv = 1.0 / jnp.maximum(aux4[:, 3:4], 1.0)
  emb = fs_ref[...] * inv
  emb_ref[...] = emb
  cent_ref[...] = aux4[:, 0:3] * inv
  h = jax.nn.relu(jnp.dot(emb, w1_ref[...],
                          preferred_element_type=jnp.float32))
  h = jax.nn.relu(jnp.dot(h, w2_ref[...],
                          preferred_element_type=jnp.float32))
  out_ref[...] = (jnp.dot(h, w3_ref[...],
                          preferred_element_type=jnp.float32) + b3_ref[...])


def kernel(features, coords, instance_ids, W1, W2, W3, b3):
  ids = instance_ids.astype(jnp.int32)

  fsum = _sc_segment_sum(features, ids, ids[::DEC])

  aux4 = pl.pallas_call(
      _tc_aux_body,
      grid=(NB2,),
      in_specs=[
          pl.BlockSpec((1, 1, BN2), lambda i: (i, 0, 0)),
          pl.BlockSpec((BN2, 3), lambda i: (i, 0)),
      ],
      out_specs=pl.BlockSpec((K, 4), lambda i: (0, 0)),
      out_shape=jax.ShapeDtypeStruct((K, 4), jnp.float32),
  )(ids.reshape(NB2, 1, BN2), coords.astype(jnp.float32))

  emb, cent, out = pl.pallas_call(
      _tc_head_body,
      out_shape=[jax.ShapeDtypeStruct((K, D), jnp.float32),
                 jax.ShapeDtypeStruct((K, 3), jnp.float32),
                 jax.ShapeDtypeStruct((K, OUT), jnp.float32)],
  )(aux4, fsum.reshape(K, D), W1, W2, W3, b3)
  return emb, cent, out
